# baseline (speedup 1.0000x reference)
import functools

import numpy as np
import jax
import jax.numpy as jnp
from jax import lax
from jax.experimental import pallas as pl
from jax.experimental.pallas import tpu as pltpu
from jax.experimental.pallas import tpu_sc as plsc

KH, KW = 3, 3
N_TAP = KH * KW

_NC, _NS = 2, 16
_NW = _NC * _NS


def _grid_offset_np(h, w):
    init = np.stack(np.meshgrid(np.arange(KH), np.arange(KW), indexing="ij"))
    init = init.reshape(-1, 2)[None, None, :, :]
    init = np.tile(init, (h, w, 1, 1)).astype(np.float32)
    off0 = int((KH - 1) / 2.0)
    off1 = int((KW - 1) / 2.0)
    grid = np.meshgrid(np.arange(-off0, h - off0), np.arange(-off1, w - off1),
                       indexing="ij")
    grid = np.stack(grid, axis=-1).astype(np.float32)[:, :, None, :]
    grid = np.tile(grid, (1, 1, N_TAP, 1))
    return grid + init


def _idx_weight_kernel(h, w, oy, ox, gy, gx, boff,
                       iy0, iy1, w00, w01, w10, w11):
    cy = jnp.clip(gy[...] + oy[...], 0.0, float(h - 1))
    cx = jnp.clip(gx[...] + ox[...], 0.0, float(w - 1))
    y0f = jnp.floor(cy)
    x0f = jnp.floor(cx)
    fy = cy - y0f
    fx = cx - x0f
    y0 = y0f.astype(jnp.int32)
    x0 = x0f.astype(jnp.int32)
    y1 = jnp.ceil(cy).astype(jnp.int32)
    base = boff[...] + x0
    iy0[...] = base + y0 * w
    iy1[...] = base + y1 * w
    gy1 = 1.0 - fy
    gx1 = 1.0 - fx
    w00[...] = gy1 * gx1
    w01[...] = gy1 * fx
    w10[...] = fy * gx1
    w11[...] = fy * fx


def _make_gather(m9, cw2, rows_per_worker, chunk):
    nchunk = rows_per_worker // chunk
    mesh = plsc.VectorSubcoreMesh(core_axis_name="c", subcore_axis_name="s")

    @functools.partial(
        pl.kernel,
        out_type=jax.ShapeDtypeStruct((2 * m9, cw2), jnp.int32),
        mesh=mesh,
        scratch_types=[
            pltpu.VMEM((chunk,), jnp.int32),
            pltpu.VMEM((chunk,), jnp.int32),
            pltpu.VMEM((chunk, cw2), jnp.int32),
            pltpu.VMEM((chunk, cw2), jnp.int32),
            pltpu.SemaphoreType.DMA,
            pltpu.SemaphoreType.DMA,
            pltpu.SemaphoreType.DMA,
            pltpu.SemaphoreType.DMA,
        ],
    )
    def gather(table_hbm, idx_hbm, out_hbm,
               idx_a, idx_b, buf_a, buf_b, gsem_a, gsem_b, wsem_a, wsem_b):
        wid = lax.axis_index("s") * _NC + lax.axis_index("c")
        base = wid * rows_per_worker
        idx_refs = (idx_a, idx_b)
        buf_refs = (buf_a, buf_b)
        gsems = (gsem_a, gsem_b)
        wsems = (wsem_a, wsem_b)

        for j in range(2):
            off = j * m9 + base
            gd = [None, None]
            wd = [None, None]

            def start_gather(ch, off=off, gd=gd):
                s = ch & 1
                pltpu.sync_copy(idx_hbm.at[pl.ds(off + ch * chunk, chunk)],
                                idx_refs[s])
                gd[s] = pltpu.async_copy(table_hbm.at[idx_refs[s]],
                                         buf_refs[s], gsems[s])

            start_gather(0)
            for ch in range(nchunk):
                s = ch & 1
                if ch + 1 < nchunk:
                    if wd[1 - s] is not None:
                        wd[1 - s].wait()
                        wd[1 - s] = None
                    start_gather(ch + 1)
                gd[s].wait()
                wd[s] = pltpu.async_copy(
                    buf_refs[s],
                    out_hbm.at[pl.ds(off + ch * chunk, chunk)],
                    wsems[s])
            for s in range(2):
                if wd[s] is not None:
                    wd[s].wait()

    return gather


def _blend_matmul_kernel(n_tap, c_in, st_ref, wg_ref, w_ref, b_ref, o_ref):
    acc = None
    for n in range(n_tap):
        s0 = st_ref[0, n]
        s1 = st_ref[1, n]
        mapped = (s0[:, :c_in] * wg_ref[n, :, 0:1]
                  + s0[:, c_in:] * wg_ref[n, :, 1:2]
                  + s1[:, :c_in] * wg_ref[n, :, 2:3]
                  + s1[:, c_in:] * wg_ref[n, :, 3:4])
        d = jnp.dot(mapped, w_ref[n], preferred_element_type=jnp.float32)
        acc = d if acc is None else acc + d
    o_ref[...] = acc + b_ref[...]


def kernel(inputs, offsets, W, b):
    bsz, h, w, c_in = inputs.shape
    n_tap, _, c_out = W.shape
    hw = h * w
    m = bsz * hw
    m9 = m * n_tap
    cw = c_in // 2

    grid = _grid_offset_np(h, w)
    gy = np.tile(grid[..., 0].transpose(2, 0, 1)[:, None], (1, bsz, 1, 1))
    gx = np.tile(grid[..., 1].transpose(2, 0, 1)[:, None], (1, bsz, 1, 1))
    boff = np.tile(np.repeat(np.arange(bsz, dtype=np.int32) * hw, hw), n_tap)

    lanes = 128
    rows128 = m9 // lanes
    gy = jnp.asarray(gy.reshape(rows128, lanes))
    gx = jnp.asarray(gx.reshape(rows128, lanes))
    boff = jnp.asarray(boff.reshape(rows128, lanes))

    off5 = offsets.reshape(bsz, h, w, n_tap, 2)
    oy = jnp.transpose(off5[..., 0], (3, 0, 1, 2)).reshape(rows128, lanes)
    ox = jnp.transpose(off5[..., 1], (3, 0, 1, 2)).reshape(rows128, lanes)

    shp_i = jax.ShapeDtypeStruct((rows128, lanes), jnp.int32)
    shp_f = jax.ShapeDtypeStruct((rows128, lanes), jnp.float32)
    iy0, iy1, w00, w01, w10, w11 = pl.pallas_call(
        functools.partial(_idx_weight_kernel, h, w),
        out_shape=(shp_i, shp_i, shp_f, shp_f, shp_f, shp_f),
    )(oy, ox, gy, gx, boff)

    idx_all = jnp.concatenate([iy0.reshape(-1), iy1.reshape(-1)])
    wgt9 = (jnp.stack([w00, w01, w10, w11], axis=-1)
            .reshape(n_tap, m, 4).astype(jnp.bfloat16))

    tp = lax.bitcast_convert_type(
        inputs.astype(jnp.bfloat16).reshape(m, cw, 2), jnp.int32)
    tp_ext = jnp.concatenate([tp, jnp.zeros((1, cw), jnp.int32)], axis=0)
    table2 = jnp.concatenate([tp_ext[:-1], tp_ext[1:]], axis=1)

    rows_per_worker = m9 // _NW
    chunk = 128
    strips = _make_gather(m9, 2 * cw, rows_per_worker, chunk)(table2, idx_all)
    strips = lax.bitcast_convert_type(strips, jnp.bfloat16).reshape(
        2, n_tap, m, 2 * c_in)

    wf = W.astype(jnp.bfloat16)
    b2 = b.reshape(1, c_out)
    lb = 128
    out = pl.pallas_call(
        functools.partial(_blend_matmul_kernel, n_tap, c_in),
        grid=(m // lb,),
        in_specs=[
            pl.BlockSpec((2, n_tap, lb, 2 * c_in), lambda i: (0, 0, i, 0)),
            pl.BlockSpec((n_tap, lb, 4), lambda i: (0, i, 0)),
            pl.BlockSpec((n_tap, c_in, c_out), lambda i: (0, 0, 0)),
            pl.BlockSpec((1, c_out), lambda i: (0, 0)),
        ],
        out_specs=pl.BlockSpec((lb, c_out), lambda i: (i, 0)),
        out_shape=jax.ShapeDtypeStruct((m, c_out), jnp.float32),
    )(strips, wgt9, wf, b2)
    return out.reshape(bsz, h, w, c_out)


# speedup vs baseline: 3.9597x; 3.9597x over previous
import functools

import numpy as np
import jax
import jax.numpy as jnp
from jax import lax
from jax.experimental import pallas as pl
from jax.experimental.pallas import tpu as pltpu
from jax.experimental.pallas import tpu_sc as plsc

KH, KW = 3, 3
N_TAP = KH * KW

_NC, _NS = 2, 16
_NW = _NC * _NS


def _grid_offset_np(h, w):
    init = np.stack(np.meshgrid(np.arange(KH), np.arange(KW), indexing="ij"))
    init = init.reshape(-1, 2)[None, None, :, :]
    init = np.tile(init, (h, w, 1, 1)).astype(np.float32)
    off0 = int((KH - 1) / 2.0)
    off1 = int((KW - 1) / 2.0)
    grid = np.meshgrid(np.arange(-off0, h - off0), np.arange(-off1, w - off1),
                       indexing="ij")
    grid = np.stack(grid, axis=-1).astype(np.float32)[:, :, None, :]
    grid = np.tile(grid, (1, 1, N_TAP, 1))
    return grid + init


def _pack_kernel(m, cw2, in_ref, out_ref):
    x = in_ref[...].astype(jnp.bfloat16)
    lo = lax.bitcast_convert_type(x[:, :cw2], jnp.uint16).astype(jnp.uint32)
    hi = lax.bitcast_convert_type(x[:, cw2:], jnp.uint16).astype(jnp.uint32)
    words = lax.bitcast_convert_type((hi << 16) | lo, jnp.int32)
    out_ref[:, :cw2] = words
    out_ref[: m - 1, cw2:] = words[1:]
    out_ref[m - 1 :, cw2:] = jnp.zeros((1, cw2), jnp.int32)


def _idx_weight_kernel(h, w, oy, ox, gy, gx, boff,
                       iy0, iy1, w00, w01, w10, w11):
    cy = jnp.clip(gy[...] + oy[...], 0.0, float(h - 1))
    cx = jnp.clip(gx[...] + ox[...], 0.0, float(w - 1))
    y0f = jnp.floor(cy)
    x0f = jnp.floor(cx)
    fy = cy - y0f
    fx = cx - x0f
    y0 = y0f.astype(jnp.int32)
    x0 = x0f.astype(jnp.int32)
    y1 = jnp.ceil(cy).astype(jnp.int32)
    base = boff[...] + x0
    iy0[...] = base + y0 * w
    iy1[...] = base + y1 * w
    gy1 = 1.0 - fy
    gx1 = 1.0 - fx
    w00[...] = gy1 * gx1
    w01[...] = gy1 * fx
    w10[...] = fy * gx1
    w11[...] = fy * fx


def _make_gather(m9, cw2, rows_per_worker, chunk):
    nchunk = rows_per_worker // chunk
    mesh = plsc.VectorSubcoreMesh(core_axis_name="c", subcore_axis_name="s")

    @functools.partial(
        pl.kernel,
        out_type=jax.ShapeDtypeStruct((2 * m9, 2 * cw2), jnp.int32),
        mesh=mesh,
        scratch_types=[
            pltpu.VMEM((chunk,), jnp.int32),
            pltpu.VMEM((chunk,), jnp.int32),
            pltpu.VMEM((chunk, 2 * cw2), jnp.int32),
            pltpu.VMEM((chunk, 2 * cw2), jnp.int32),
            pltpu.SemaphoreType.DMA,
            pltpu.SemaphoreType.DMA,
            pltpu.SemaphoreType.DMA,
            pltpu.SemaphoreType.DMA,
        ],
    )
    def gather(table_hbm, idx_hbm, out_hbm,
               idx_a, idx_b, buf_a, buf_b, gsem_a, gsem_b, wsem_a, wsem_b):
        wid = lax.axis_index("s") * _NC + lax.axis_index("c")
        base = wid * rows_per_worker
        idx_refs = (idx_a, idx_b)
        buf_refs = (buf_a, buf_b)
        gsems = (gsem_a, gsem_b)
        wsems = (wsem_a, wsem_b)

        for j in range(2):
            off = j * m9 + base
            gd = [None, None]
            wd = [None, None]

            def start_gather(ch, off=off, gd=gd):
                s = ch & 1
                pltpu.sync_copy(idx_hbm.at[pl.ds(off + ch * chunk, chunk)],
                                idx_refs[s])
                gd[s] = pltpu.async_copy(table_hbm.at[idx_refs[s]],
                                         buf_refs[s], gsems[s])

            start_gather(0)
            for ch in range(nchunk):
                s = ch & 1
                if ch + 1 < nchunk:
                    if wd[1 - s] is not None:
                        wd[1 - s].wait()
                        wd[1 - s] = None
                    start_gather(ch + 1)
                gd[s].wait()
                wd[s] = pltpu.async_copy(
                    buf_refs[s],
                    out_hbm.at[pl.ds(off + ch * chunk, chunk)],
                    wsems[s])
            for s in range(2):
                if wd[s] is not None:
                    wd[s].wait()

    return gather


def _blend_matmul_kernel(n_tap, cw2, lb, st_ref, wg_ref, wa_ref, wb_ref,
                         b_ref, o_ref):
    acc_a = None
    acc_b = None
    for n in range(n_tap):
        u0 = pltpu.bitcast(st_ref[0, n], jnp.bfloat16)
        u1 = pltpu.bitcast(st_ref[1, n], jnp.bfloat16)
        mapped = (u0[:, :cw2] * wg_ref[n, :, 0:1]
                  + u0[:, cw2:] * wg_ref[n, :, 1:2]
                  + u1[:, :cw2] * wg_ref[n, :, 2:3]
                  + u1[:, cw2:] * wg_ref[n, :, 3:4])
        da = jnp.dot(mapped, wa_ref[n], preferred_element_type=jnp.float32)
        db = jnp.dot(mapped, wb_ref[n], preferred_element_type=jnp.float32)
        acc_a = da if acc_a is None else acc_a + da
        acc_b = db if acc_b is None else acc_b + db
    res = acc_a + pltpu.roll(acc_b, 2 * lb - 1, 0)
    o_ref[...] = res + b_ref[...]


def kernel(inputs, offsets, W, b):
    bsz, h, w, c_in = inputs.shape
    n_tap, _, c_out = W.shape
    hw = h * w
    m = bsz * hw
    m9 = m * n_tap
    cw2 = c_in // 2

    grid = _grid_offset_np(h, w)
    gy = np.tile(grid[..., 0].transpose(2, 0, 1)[:, None], (1, bsz, 1, 1))
    gx = np.tile(grid[..., 1].transpose(2, 0, 1)[:, None], (1, bsz, 1, 1))
    boff = np.tile(np.repeat(np.arange(bsz, dtype=np.int32) * hw, hw), n_tap)

    lanes = 128
    rows128 = m9 // lanes
    gy = jnp.asarray(gy.reshape(rows128, lanes))
    gx = jnp.asarray(gx.reshape(rows128, lanes))
    boff = jnp.asarray(boff.reshape(rows128, lanes))

    off5 = offsets.reshape(bsz, h, w, n_tap, 2)
    oy = jnp.transpose(off5[..., 0], (3, 0, 1, 2)).reshape(rows128, lanes)
    ox = jnp.transpose(off5[..., 1], (3, 0, 1, 2)).reshape(rows128, lanes)

    table2 = pl.pallas_call(
        functools.partial(_pack_kernel, m, cw2),
        out_shape=jax.ShapeDtypeStruct((m, 2 * cw2), jnp.int32),
    )(inputs.reshape(m, c_in))

    shp_i = jax.ShapeDtypeStruct((rows128, lanes), jnp.int32)
    shp_f = jax.ShapeDtypeStruct((rows128, lanes), jnp.float32)
    iy0, iy1, w00, w01, w10, w11 = pl.pallas_call(
        functools.partial(_idx_weight_kernel, h, w),
        out_shape=(shp_i, shp_i, shp_f, shp_f, shp_f, shp_f),
    )(oy, ox, gy, gx, boff)

    idx_all = jnp.concatenate([iy0.reshape(-1), iy1.reshape(-1)])
    wgt9 = (jnp.stack([w00, w01, w10, w11], axis=-1)
            .reshape(n_tap, m, 1, 4).astype(jnp.bfloat16))
    wgt9 = jnp.broadcast_to(wgt9, (n_tap, m, 2, 4)).reshape(n_tap, 2 * m, 4)

    rows_per_worker = m9 // _NW
    chunk = 128
    strips = _make_gather(m9, cw2, rows_per_worker, chunk)(table2, idx_all)
    strips = strips.reshape(2, n_tap, m, 2 * cw2)

    wf = W.astype(jnp.bfloat16)
    wa = wf[:, :cw2, :]
    wb = wf[:, cw2:, :]
    b2 = b.reshape(1, c_out)
    lb = 128
    out2 = pl.pallas_call(
        functools.partial(_blend_matmul_kernel, n_tap, cw2, lb),
        grid=(m // lb,),
        in_specs=[
            pl.BlockSpec((2, n_tap, lb, 2 * cw2), lambda i: (0, 0, i, 0)),
            pl.BlockSpec((n_tap, 2 * lb, 4), lambda i: (0, i, 0)),
            pl.BlockSpec((n_tap, cw2, c_out), lambda i: (0, 0, 0)),
            pl.BlockSpec((n_tap, cw2, c_out), lambda i: (0, 0, 0)),
            pl.BlockSpec((1, c_out), lambda i: (0, 0)),
        ],
        out_specs=pl.BlockSpec((2 * lb, c_out), lambda i: (i, 0)),
        out_shape=jax.ShapeDtypeStruct((2 * m, c_out), jnp.float32),
    )(strips, wgt9, wa, wb, b2)
    out = out2.reshape(m, 2, c_out)[:, 0, :]
    return out.reshape(bsz, h, w, c_out)


# speedup vs baseline: 4.3865x; 1.1078x over previous
import functools

import numpy as np
import jax
import jax.numpy as jnp
from jax import lax
from jax.experimental import pallas as pl
from jax.experimental.pallas import tpu as pltpu
from jax.experimental.pallas import tpu_sc as plsc

KH, KW = 3, 3
N_TAP = KH * KW

_NC, _NS = 2, 16
_NW = _NC * _NS


def _grid_offset_np(h, w):
    init = np.stack(np.meshgrid(np.arange(KH), np.arange(KW), indexing="ij"))
    init = init.reshape(-1, 2)[None, None, :, :]
    init = np.tile(init, (h, w, 1, 1)).astype(np.float32)
    off0 = int((KH - 1) / 2.0)
    off1 = int((KW - 1) / 2.0)
    grid = np.meshgrid(np.arange(-off0, h - off0), np.arange(-off1, w - off1),
                       indexing="ij")
    grid = np.stack(grid, axis=-1).astype(np.float32)[:, :, None, :]
    grid = np.tile(grid, (1, 1, N_TAP, 1))
    return grid + init


def _prep_kernel(h, w, m, cw2,
                 img, oy, ox, gy, gx, boff,
                 table_out, idx_out, w00, w01, w10, w11):
    x = img[...].astype(jnp.bfloat16)
    lo = lax.bitcast_convert_type(x[:, :cw2], jnp.uint16).astype(jnp.uint32)
    hi = lax.bitcast_convert_type(x[:, cw2:], jnp.uint16).astype(jnp.uint32)
    words = lax.bitcast_convert_type((hi << 16) | lo, jnp.int32)
    table_out[:, :cw2] = words
    table_out[: m - 1, cw2:] = words[1:]
    table_out[m - 1 :, cw2:] = jnp.zeros((1, cw2), jnp.int32)

    cy = jnp.clip(gy[...] + oy[...], 0.0, float(h - 1))
    cx = jnp.clip(gx[...] + ox[...], 0.0, float(w - 1))
    y0f = jnp.floor(cy)
    x0f = jnp.floor(cx)
    fy = cy - y0f
    fx = cx - x0f
    y0 = y0f.astype(jnp.int32)
    x0 = x0f.astype(jnp.int32)
    y1 = jnp.ceil(cy).astype(jnp.int32)
    base = boff[...] + x0
    idx_out[0] = base + y0 * w
    idx_out[1] = base + y1 * w
    gy1 = 1.0 - fy
    gx1 = 1.0 - fx
    w00[...] = gy1 * gx1
    w01[...] = gy1 * fx
    w10[...] = fy * gx1
    w11[...] = fy * fx


def _make_gather(m9, cw2, rows_per_worker, chunk):
    nchunk = rows_per_worker // chunk
    mesh = plsc.VectorSubcoreMesh(core_axis_name="c", subcore_axis_name="s")

    @functools.partial(
        pl.kernel,
        out_type=jax.ShapeDtypeStruct((2 * m9, 2 * cw2), jnp.int32),
        mesh=mesh,
        scratch_types=[
            pltpu.VMEM((chunk,), jnp.int32),
            pltpu.VMEM((chunk,), jnp.int32),
            pltpu.VMEM((chunk, 2 * cw2), jnp.int32),
            pltpu.VMEM((chunk, 2 * cw2), jnp.int32),
            pltpu.SemaphoreType.DMA,
            pltpu.SemaphoreType.DMA,
            pltpu.SemaphoreType.DMA,
            pltpu.SemaphoreType.DMA,
        ],
    )
    def gather(table_hbm, idx_hbm, out_hbm,
               idx_a, idx_b, buf_a, buf_b, gsem_a, gsem_b, wsem_a, wsem_b):
        wid = lax.axis_index("s") * _NC + lax.axis_index("c")
        base = wid * rows_per_worker
        idx_refs = (idx_a, idx_b)
        buf_refs = (buf_a, buf_b)
        gsems = (gsem_a, gsem_b)
        wsems = (wsem_a, wsem_b)

        for j in range(2):
            off = j * m9 + base
            gd = [None, None]
            wd = [None, None]

            def start_gather(ch, off=off, gd=gd):
                s = ch & 1
                pltpu.sync_copy(idx_hbm.at[pl.ds(off + ch * chunk, chunk)],
                                idx_refs[s])
                gd[s] = pltpu.async_copy(table_hbm.at[idx_refs[s]],
                                         buf_refs[s], gsems[s])

            start_gather(0)
            for ch in range(nchunk):
                s = ch & 1
                if ch + 1 < nchunk:
                    if wd[1 - s] is not None:
                        wd[1 - s].wait()
                        wd[1 - s] = None
                    start_gather(ch + 1)
                gd[s].wait()
                wd[s] = pltpu.async_copy(
                    buf_refs[s],
                    out_hbm.at[pl.ds(off + ch * chunk, chunk)],
                    wsems[s])
            for s in range(2):
                if wd[s] is not None:
                    wd[s].wait()

    return gather


def _blend_matmul_kernel(n_tap, cw2, lb, st_ref, wg_ref, wa_ref, wb_ref,
                         b_ref, o_ref):
    acc_a = None
    acc_b = None
    for n in range(n_tap):
        u0 = pltpu.bitcast(st_ref[0, n], jnp.bfloat16)
        u1 = pltpu.bitcast(st_ref[1, n], jnp.bfloat16)
        mapped = (u0[:, :cw2] * wg_ref[n, :, 0:1]
                  + u0[:, cw2:] * wg_ref[n, :, 1:2]
                  + u1[:, :cw2] * wg_ref[n, :, 2:3]
                  + u1[:, cw2:] * wg_ref[n, :, 3:4])
        da = jnp.dot(mapped, wa_ref[n], preferred_element_type=jnp.float32)
        db = jnp.dot(mapped, wb_ref[n], preferred_element_type=jnp.float32)
        acc_a = da if acc_a is None else acc_a + da
        acc_b = db if acc_b is None else acc_b + db
    o_ref[...] = acc_a + pltpu.roll(acc_b, 2 * lb - 1, 0) + b_ref[...]


def kernel(inputs, offsets, W, b):
    bsz, h, w, c_in = inputs.shape
    n_tap, _, c_out = W.shape
    hw = h * w
    m = bsz * hw
    m9 = m * n_tap
    cw2 = c_in // 2

    grid = _grid_offset_np(h, w)
    gy = np.tile(grid[..., 0].transpose(2, 0, 1)[:, None], (1, bsz, 1, 1))
    gx = np.tile(grid[..., 1].transpose(2, 0, 1)[:, None], (1, bsz, 1, 1))
    boff = np.tile(np.repeat(np.arange(bsz, dtype=np.int32) * hw, hw), n_tap)

    lanes = 128
    rows128 = m9 // lanes
    gy = jnp.asarray(gy.reshape(rows128, lanes))
    gx = jnp.asarray(gx.reshape(rows128, lanes))
    boff = jnp.asarray(boff.reshape(rows128, lanes))

    off5 = offsets.reshape(bsz, h, w, n_tap, 2)
    oy = jnp.transpose(off5[..., 0], (3, 0, 1, 2)).reshape(rows128, lanes)
    ox = jnp.transpose(off5[..., 1], (3, 0, 1, 2)).reshape(rows128, lanes)

    shp_f = jax.ShapeDtypeStruct((rows128, lanes), jnp.float32)
    table2, idx2, w00, w01, w10, w11 = pl.pallas_call(
        functools.partial(_prep_kernel, h, w, m, cw2),
        out_shape=(
            jax.ShapeDtypeStruct((m, 2 * cw2), jnp.int32),
            jax.ShapeDtypeStruct((2, rows128, lanes), jnp.int32),
            shp_f, shp_f, shp_f, shp_f,
        ),
    )(inputs.reshape(m, c_in), oy, ox, gy, gx, boff)

    idx_all = idx2.reshape(2 * m9)
    wgt9 = (jnp.stack([w00, w01, w10, w11], axis=-1)
            .reshape(n_tap, m, 1, 4).astype(jnp.bfloat16))
    wgt9 = jnp.broadcast_to(wgt9, (n_tap, m, 2, 4)).reshape(n_tap, 2 * m, 4)

    rows_per_worker = m9 // _NW
    chunk = 128
    strips = _make_gather(m9, cw2, rows_per_worker, chunk)(table2, idx_all)
    strips = strips.reshape(2, n_tap, m, 2 * cw2)

    wf = W.astype(jnp.bfloat16)
    wa = wf[:, :cw2, :]
    wb = wf[:, cw2:, :]
    b2 = b.reshape(1, c_out)
    lb = 256
    out = pl.pallas_call(
        functools.partial(_blend_matmul_kernel, n_tap, cw2, lb),
        grid=(m // lb,),
        in_specs=[
            pl.BlockSpec((2, n_tap, lb, 2 * cw2), lambda i: (0, 0, i, 0)),
            pl.BlockSpec((n_tap, 2 * lb, 4), lambda i: (0, i, 0)),
            pl.BlockSpec((n_tap, cw2, c_out), lambda i: (0, 0, 0)),
            pl.BlockSpec((n_tap, cw2, c_out), lambda i: (0, 0, 0)),
            pl.BlockSpec((1, c_out), lambda i: (0, 0)),
        ],
        out_specs=pl.BlockSpec((2 * lb, c_out), lambda i: (i, 0)),
        out_shape=jax.ShapeDtypeStruct((2 * m, c_out), jnp.float32),
    )(strips, wgt9, wa, wb, b2)
    out = out.reshape(m, 2, c_out)[:, 0, :]
    return out.reshape(bsz, h, w, c_out)


# speedup vs baseline: 4.6298x; 1.0555x over previous
import functools

import numpy as np
import jax
import jax.numpy as jnp
from jax import lax
from jax.experimental import pallas as pl
from jax.experimental.pallas import tpu as pltpu
from jax.experimental.pallas import tpu_sc as plsc

KH, KW = 3, 3
N_TAP = KH * KW

_NC, _NS = 2, 16
_NW = _NC * _NS


def _grid_offset_np(h, w):
    init = np.stack(np.meshgrid(np.arange(KH), np.arange(KW), indexing="ij"))
    init = init.reshape(-1, 2)[None, None, :, :]
    init = np.tile(init, (h, w, 1, 1)).astype(np.float32)
    off0 = int((KH - 1) / 2.0)
    off1 = int((KW - 1) / 2.0)
    grid = np.meshgrid(np.arange(-off0, h - off0), np.arange(-off1, w - off1),
                       indexing="ij")
    grid = np.stack(grid, axis=-1).astype(np.float32)[:, :, None, :]
    grid = np.tile(grid, (1, 1, N_TAP, 1))
    return grid + init


def _prep_kernel(h, w, m, cw2,
                 img, oy, ox, gy, gx, boff,
                 table_out, idx_out, w00, w01, w10, w11):
    x = img[...].astype(jnp.bfloat16)
    lo = lax.bitcast_convert_type(x[:, :cw2], jnp.uint16).astype(jnp.uint32)
    hi = lax.bitcast_convert_type(x[:, cw2:], jnp.uint16).astype(jnp.uint32)
    words = lax.bitcast_convert_type((hi << 16) | lo, jnp.int32)
    table_out[:, :cw2] = words
    table_out[: m - 1, cw2:] = words[1:]
    table_out[m - 1 :, cw2:] = jnp.zeros((1, cw2), jnp.int32)

    cy = jnp.clip(gy[...] + oy[...], 0.0, float(h - 1))
    cx = jnp.clip(gx[...] + ox[...], 0.0, float(w - 1))
    y0f = jnp.floor(cy)
    x0f = jnp.floor(cx)
    fy = cy - y0f
    fx = cx - x0f
    y0 = y0f.astype(jnp.int32)
    x0 = x0f.astype(jnp.int32)
    y1 = jnp.ceil(cy).astype(jnp.int32)
    base = boff[...] + x0
    idx_out[0] = base + y0 * w
    idx_out[1] = base + y1 * w
    gy1 = 1.0 - fy
    gx1 = 1.0 - fx
    w00[...] = gy1 * gx1
    w01[...] = gy1 * fx
    w10[...] = fy * gx1
    w11[...] = fy * fx


def _make_gather(m9, cw2, rows_per_worker, chunk):
    nchunk = rows_per_worker // chunk
    mesh = plsc.VectorSubcoreMesh(core_axis_name="c", subcore_axis_name="s")

    @functools.partial(
        pl.kernel,
        out_type=jax.ShapeDtypeStruct((2 * m9, 2 * cw2), jnp.int32),
        mesh=mesh,
        scratch_types=[
            pltpu.VMEM((chunk,), jnp.int32),
            pltpu.VMEM((chunk,), jnp.int32),
            pltpu.VMEM((chunk, 2 * cw2), jnp.int32),
            pltpu.VMEM((chunk, 2 * cw2), jnp.int32),
            pltpu.SemaphoreType.DMA,
            pltpu.SemaphoreType.DMA,
            pltpu.SemaphoreType.DMA,
            pltpu.SemaphoreType.DMA,
        ],
    )
    def gather(table_hbm, idx_hbm, out_hbm,
               idx_a, idx_b, buf_a, buf_b, gsem_a, gsem_b, wsem_a, wsem_b):
        wid = lax.axis_index("s") * _NC + lax.axis_index("c")
        base = wid * rows_per_worker
        idx_refs = (idx_a, idx_b)
        buf_refs = (buf_a, buf_b)
        gsems = (gsem_a, gsem_b)
        wsems = (wsem_a, wsem_b)

        for j in range(2):
            off = j * m9 + base
            gd = [None, None]
            wd = [None, None]

            def start_gather(ch, off=off, gd=gd):
                s = ch & 1
                pltpu.sync_copy(idx_hbm.at[pl.ds(off + ch * chunk, chunk)],
                                idx_refs[s])
                gd[s] = pltpu.async_copy(table_hbm.at[idx_refs[s]],
                                         buf_refs[s], gsems[s])

            start_gather(0)
            for ch in range(nchunk):
                s = ch & 1
                if ch + 1 < nchunk:
                    if wd[1 - s] is not None:
                        wd[1 - s].wait()
                        wd[1 - s] = None
                    start_gather(ch + 1)
                gd[s].wait()
                wd[s] = pltpu.async_copy(
                    buf_refs[s],
                    out_hbm.at[pl.ds(off + ch * chunk, chunk)],
                    wsems[s])
            for s in range(2):
                if wd[s] is not None:
                    wd[s].wait()

    return gather


def _blend_matmul_kernel(n_tap, cw2, lb, st_ref, wg_ref, wa_ref, wb_ref,
                         b_ref, o_ref):
    acc_a = None
    acc_b = None
    for n in range(n_tap):
        u0 = pltpu.bitcast(st_ref[0, n], jnp.bfloat16)
        u1 = pltpu.bitcast(st_ref[1, n], jnp.bfloat16)
        mapped = (u0[:, :cw2] * wg_ref[n, :, 0:1]
                  + u0[:, cw2:] * wg_ref[n, :, 1:2]
                  + u1[:, :cw2] * wg_ref[n, :, 2:3]
                  + u1[:, cw2:] * wg_ref[n, :, 3:4])
        da = jnp.dot(mapped, wa_ref[n], preferred_element_type=jnp.float32)
        db = jnp.dot(mapped, wb_ref[n], preferred_element_type=jnp.float32)
        acc_a = da if acc_a is None else acc_a + da
        acc_b = db if acc_b is None else acc_b + db
    o_ref[...] = acc_a + pltpu.roll(acc_b, 2 * lb - 1, 0) + b_ref[...]


def kernel(inputs, offsets, W, b):
    bsz, h, w, c_in = inputs.shape
    n_tap, _, c_out = W.shape
    hw = h * w
    m = bsz * hw
    m9 = m * n_tap
    cw2 = c_in // 2

    grid = _grid_offset_np(h, w)
    gy = np.tile(grid[..., 0].transpose(2, 0, 1)[:, None], (1, bsz, 1, 1))
    gx = np.tile(grid[..., 1].transpose(2, 0, 1)[:, None], (1, bsz, 1, 1))
    boff = np.tile(np.repeat(np.arange(bsz, dtype=np.int32) * hw, hw), n_tap)

    lanes = 128
    rows128 = m9 // lanes
    gy = jnp.asarray(gy.reshape(rows128, lanes))
    gx = jnp.asarray(gx.reshape(rows128, lanes))
    boff = jnp.asarray(boff.reshape(rows128, lanes))

    off5 = offsets.reshape(bsz, h, w, n_tap, 2)
    oy = jnp.transpose(off5[..., 0], (3, 0, 1, 2)).reshape(rows128, lanes)
    ox = jnp.transpose(off5[..., 1], (3, 0, 1, 2)).reshape(rows128, lanes)

    shp_f = jax.ShapeDtypeStruct((rows128, lanes), jnp.float32)
    table2, idx2, w00, w01, w10, w11 = pl.pallas_call(
        functools.partial(_prep_kernel, h, w, m, cw2),
        out_shape=(
            jax.ShapeDtypeStruct((m, 2 * cw2), jnp.int32),
            jax.ShapeDtypeStruct((2, rows128, lanes), jnp.int32),
            shp_f, shp_f, shp_f, shp_f,
        ),
    )(inputs.reshape(m, c_in), oy, ox, gy, gx, boff)

    wgt9 = (jnp.stack([w00, w01, w10, w11], axis=-1)
            .reshape(n_tap, m, 1, 4).astype(jnp.bfloat16))
    wgt9 = jnp.broadcast_to(wgt9, (n_tap, m, 2, 4)).reshape(n_tap, 2 * m, 4)

    wf = W.astype(jnp.bfloat16)
    wa = wf[:, :cw2, :]
    wb = wf[:, cw2:, :]
    b2 = b.reshape(1, c_out)
    lb = 256

    nseg = 2
    m2 = m // nseg
    m9s = m2 * n_tap
    rows_per_worker = m9s // _NW
    chunk = 96
    sc_fn = _make_gather(m9s, cw2, rows_per_worker, chunk)
    idx3 = idx2.reshape(2, n_tap, m)

    outs = []
    for seg in range(nseg):
        idx_seg = idx3[:, :, seg * m2:(seg + 1) * m2].reshape(2 * m9s)
        strips = sc_fn(table2, idx_seg).reshape(2, n_tap, m2, 2 * cw2)
        wg_seg = wgt9[:, 2 * seg * m2:2 * (seg + 1) * m2]
        out_h = pl.pallas_call(
            functools.partial(_blend_matmul_kernel, n_tap, cw2, lb),
            grid=(m2 // lb,),
            in_specs=[
                pl.BlockSpec((2, n_tap, lb, 2 * cw2), lambda i: (0, 0, i, 0)),
                pl.BlockSpec((n_tap, 2 * lb, 4), lambda i: (0, i, 0)),
                pl.BlockSpec((n_tap, cw2, c_out), lambda i: (0, 0, 0)),
                pl.BlockSpec((n_tap, cw2, c_out), lambda i: (0, 0, 0)),
                pl.BlockSpec((1, c_out), lambda i: (0, 0)),
            ],
            out_specs=pl.BlockSpec((2 * lb, c_out), lambda i: (i, 0)),
            out_shape=jax.ShapeDtypeStruct((2 * m2, c_out), jnp.float32),
        )(strips, wg_seg, wa, wb, b2)
        outs.append(out_h.reshape(m2, 2, c_out)[:, 0, :])
    out = jnp.concatenate(outs)
    return out.reshape(bsz, h, w, c_out)
